# Initial kernel scaffold; baseline (speedup 1.0000x reference)
import jax
import jax.numpy as jnp
from jax.experimental import pallas as pl


def kernel(cncpt_ids, edge_index, emb_table, W1, b1, W2, b2):
    raise NotImplementedError("write your pallas kernel here")


# speedup vs baseline: 4.8314x; 4.8314x over previous
import functools

import jax
import jax.numpy as jnp
from jax import lax
from jax.experimental import pallas as pl
from jax.experimental.pallas import tpu as pltpu
from jax.experimental.pallas import tpu_sc as plsc

N_NODES = 10000
N_EDGES = 160000
D_IN = 256
HALF = 128

NC = 2
NS = 16
NW = NC * NS

NPAD = 10240
ROWS_W = NPAD // NW
GCHUNK = 80
GCHUNKS = ROWS_W // GCHUNK

EPT = N_EDGES // NS
K = 80
CPT = EPT // K
NAGG = 10240
ZROWS = NAGG // NS


def _sc_mesh():
    return plsc.VectorSubcoreMesh(core_axis_name="c", subcore_axis_name="s")


def _emb_gather_body(ids_hbm, emb_hbm, lo_hbm, hi_hbm, idx_v, stag, sem):
    c = lax.axis_index("c")
    s = lax.axis_index("s")
    wid = s * NC + c

    def chunk(j, carry):
        base = wid * ROWS_W + j * GCHUNK
        pltpu.sync_copy(ids_hbm.at[pl.ds(base, GCHUNK)], idx_v)
        pltpu.async_copy(emb_hbm.at[idx_v], stag, sem).wait()
        pltpu.sync_copy(stag.at[:, pl.ds(0, HALF)], lo_hbm.at[pl.ds(base, GCHUNK)])
        pltpu.sync_copy(stag.at[:, pl.ds(HALF, HALF)], hi_hbm.at[pl.ds(base, GCHUNK)])
        return carry

    lax.fori_loop(0, GCHUNKS, chunk, 0)


def _sc_emb_gather(ids_pad, emb_table):
    out = (
        jax.ShapeDtypeStruct((NPAD, HALF), jnp.float32),
        jax.ShapeDtypeStruct((NPAD, HALF), jnp.float32),
    )
    scratch = [
        pltpu.VMEM((GCHUNK,), jnp.int32),
        pltpu.VMEM((GCHUNK, D_IN), jnp.float32),
        pltpu.SemaphoreType.DMA,
    ]
    f = pl.kernel(
        _emb_gather_body,
        out_type=out,
        mesh=_sc_mesh(),
        scratch_types=scratch,
    )
    return f(ids_pad, emb_table)


def _edge_agg_body(tlo, thi, e3, zeros_hbm, lo_out, hi_out,
                   sslab, dslab, stag, acc, sem):
    c = lax.axis_index("c")
    s = lax.axis_index("s")

    pltpu.sync_copy(zeros_hbm, acc.at[pl.ds(s * ZROWS, ZROWS)])
    pltpu.sync_copy(e3.at[0, s], sslab)
    pltpu.sync_copy(e3.at[1, s], dslab)
    plsc.subcore_barrier()

    def run(table, out):
        def chunk(j, carry):
            pltpu.async_copy(table.at[sslab.at[j]], stag, sem).wait()
            pltpu.sync_copy(stag, acc.at[dslab.at[j]], add=True)
            return carry

        lax.fori_loop(0, CPT, chunk, 0)
        plsc.subcore_barrier()
        pltpu.sync_copy(acc.at[pl.ds(s * ZROWS, ZROWS)],
                        out.at[pl.ds(s * ZROWS, ZROWS)])

    @pl.when(c == 0)
    def _():
        run(tlo, lo_out)

    @pl.when(c == 1)
    def _():
        run(thi, hi_out)


def _sc_edge_agg(tlo, thi, e3, zeros):
    out = (
        jax.ShapeDtypeStruct((NAGG, HALF), jnp.float32),
        jax.ShapeDtypeStruct((NAGG, HALF), jnp.float32),
    )
    scratch = [
        pltpu.VMEM((CPT, K), jnp.int32),
        pltpu.VMEM((CPT, K), jnp.int32),
        pltpu.VMEM((K, HALF), jnp.float32),
        pltpu.VMEM_SHARED((NAGG, HALF), jnp.float32),
        pltpu.SemaphoreType.DMA,
    ]
    f = pl.kernel(
        _edge_agg_body,
        out_type=out,
        mesh=_sc_mesh(),
        scratch_types=scratch,
    )
    return f(tlo, thi, e3, zeros)


RB = 400


def _linear_relu_body_split(lo_ref, hi_ref, w_ref, b_ref, olo_ref, ohi_ref):
    x = jnp.concatenate([lo_ref[...], hi_ref[...]], axis=1)
    y = jnp.dot(x, w_ref[...], preferred_element_type=jnp.float32) + b_ref[...]
    y = jnp.maximum(y, 0.0)
    olo_ref[...] = y[:, :HALF]
    ohi_ref[...] = y[:, HALF:]


def _tc_linear_relu_split(lo, hi, W, b2d):
    grid = (N_NODES // RB,)
    return pl.pallas_call(
        _linear_relu_body_split,
        grid=grid,
        in_specs=[
            pl.BlockSpec((RB, HALF), lambda i: (i, 0)),
            pl.BlockSpec((RB, HALF), lambda i: (i, 0)),
            pl.BlockSpec((D_IN, D_IN), lambda i: (0, 0)),
            pl.BlockSpec((1, D_IN), lambda i: (0, 0)),
        ],
        out_specs=(
            pl.BlockSpec((RB, HALF), lambda i: (i, 0)),
            pl.BlockSpec((RB, HALF), lambda i: (i, 0)),
        ),
        out_shape=(
            jax.ShapeDtypeStruct((N_NODES, HALF), jnp.float32),
            jax.ShapeDtypeStruct((N_NODES, HALF), jnp.float32),
        ),
    )(lo, hi, W, b2d)


def _linear_relu_body_full(lo_ref, hi_ref, w_ref, b_ref, o_ref):
    x = jnp.concatenate([lo_ref[...], hi_ref[...]], axis=1)
    y = jnp.dot(x, w_ref[...], preferred_element_type=jnp.float32) + b_ref[...]
    o_ref[...] = jnp.maximum(y, 0.0)


def _tc_linear_relu_full(lo, hi, W, b2d):
    grid = (N_NODES // RB,)
    return pl.pallas_call(
        _linear_relu_body_full,
        grid=grid,
        in_specs=[
            pl.BlockSpec((RB, HALF), lambda i: (i, 0)),
            pl.BlockSpec((RB, HALF), lambda i: (i, 0)),
            pl.BlockSpec((D_IN, D_IN), lambda i: (0, 0)),
            pl.BlockSpec((1, D_IN), lambda i: (0, 0)),
        ],
        out_specs=pl.BlockSpec((RB, D_IN), lambda i: (i, 0)),
        out_shape=jax.ShapeDtypeStruct((N_NODES, D_IN), jnp.float32),
    )(lo, hi, W, b2d)


def kernel(cncpt_ids, edge_index, emb_table, W1, b1, W2, b2):
    ids = cncpt_ids.astype(jnp.int32)
    ids_pad = jnp.concatenate(
        [ids, jnp.zeros((NPAD - N_NODES,), jnp.int32)])
    e3 = edge_index.astype(jnp.int32).reshape(2, NS, CPT, K)
    zeros = jnp.zeros((ZROWS, HALF), jnp.float32)

    flo, fhi = _sc_emb_gather(ids_pad, emb_table)
    a1lo, a1hi = _sc_edge_agg(flo, fhi, e3, zeros)
    h1lo, h1hi = _tc_linear_relu_split(a1lo, a1hi, W1, b1.reshape(1, D_IN))
    a2lo, a2hi = _sc_edge_agg(h1lo, h1hi, e3, zeros)
    h2 = _tc_linear_relu_full(a2lo, a2hi, W2, b2.reshape(1, D_IN))
    return h2


# speedup vs baseline: 6.5084x; 1.3471x over previous
import functools

import jax
import jax.numpy as jnp
from jax import lax
from jax.experimental import pallas as pl
from jax.experimental.pallas import tpu as pltpu
from jax.experimental.pallas import tpu_sc as plsc

N_NODES = 10000
N_EDGES = 160000
D_IN = 256
HALF = 128

NC = 2
NS = 16
NW = NC * NS

NPAD = 10240
ROWS_W = NPAD // NW
GCHUNK = 80
GCHUNKS = ROWS_W // GCHUNK

EPT = N_EDGES // NS
K = 80
CPT = EPT // K
NAGG = 10240
ZROWS = NAGG // NS


def _sc_mesh():
    return plsc.VectorSubcoreMesh(core_axis_name="c", subcore_axis_name="s")


def _emb_gather_body(ids_hbm, emb_hbm, lo_hbm, hi_hbm, idx_v, stag, sem):
    c = lax.axis_index("c")
    s = lax.axis_index("s")
    wid = s * NC + c

    def chunk(j, carry):
        base = wid * ROWS_W + j * GCHUNK
        pltpu.sync_copy(ids_hbm.at[pl.ds(base, GCHUNK)], idx_v)
        pltpu.async_copy(emb_hbm.at[idx_v], stag, sem).wait()
        pltpu.sync_copy(stag.at[:, pl.ds(0, HALF)], lo_hbm.at[pl.ds(base, GCHUNK)])
        pltpu.sync_copy(stag.at[:, pl.ds(HALF, HALF)], hi_hbm.at[pl.ds(base, GCHUNK)])
        return carry

    lax.fori_loop(0, GCHUNKS, chunk, 0)


def _sc_emb_gather(ids_pad, emb_table):
    out = (
        jax.ShapeDtypeStruct((NPAD, HALF), jnp.float32),
        jax.ShapeDtypeStruct((NPAD, HALF), jnp.float32),
    )
    scratch = [
        pltpu.VMEM((GCHUNK,), jnp.int32),
        pltpu.VMEM((GCHUNK, D_IN), jnp.float32),
        pltpu.SemaphoreType.DMA,
    ]
    f = pl.kernel(
        _emb_gather_body,
        out_type=out,
        mesh=_sc_mesh(),
        scratch_types=scratch,
    )
    return f(ids_pad, emb_table)


NBUF = 4
NROUNDS = CPT // NBUF
REM = CPT - NROUNDS * NBUF


def _edge_agg_body(tlo, thi, e4, zeros_hbm, lo_out, hi_out,
                   ering, stags, acc, *sems):
    isems = [sems[b] for b in range(NBUF)]
    gsems = [sems[NBUF + b] for b in range(NBUF)]
    ssems = [sems[2 * NBUF + b] for b in range(NBUF)]
    c = lax.axis_index("c")
    s = lax.axis_index("s")

    pltpu.sync_copy(zeros_hbm, acc.at[pl.ds(s * ZROWS, ZROWS)])
    plsc.subcore_barrier()

    def run(table, out):
        def issue_idx(b, j):
            pltpu.async_copy(e4.at[s, j], ering.at[b], isems[b])

        def wait_idx(b):
            pltpu.make_async_copy(e4.at[s, 0], ering.at[b], isems[b]).wait()

        def issue_gather(b):
            pltpu.async_copy(table.at[ering.at[b, 0]], stags.at[b], gsems[b])

        def wait_gather(b):
            pltpu.make_async_copy(
                table.at[ering.at[b, 0]], stags.at[b], gsems[b]).wait()

        def issue_scatter(b):
            pltpu.async_copy(stags.at[b], acc.at[ering.at[b, 1]],
                             ssems[b], add=True)

        def wait_scatter(b):
            pltpu.make_async_copy(
                stags.at[b], acc.at[ering.at[b, 1]], ssems[b]).wait()

        for b in range(NBUF):
            issue_idx(b, b)
        for b in range(NBUF):
            wait_idx(b)
            issue_gather(b)

        def outer(t, carry):
            for b in range(NBUF):
                wait_gather(b)
                issue_scatter(b)
            for b in range(NBUF):
                j2 = jnp.minimum((t + 1) * NBUF + b, CPT - 1)
                wait_scatter(b)
                issue_idx(b, j2)
            for b in range(NBUF):
                wait_idx(b)
                issue_gather(b)
            return carry

        lax.fori_loop(0, NROUNDS - 1, outer, 0)
        for b in range(NBUF):
            wait_gather(b)
            issue_scatter(b)
        for b in range(NBUF):
            wait_scatter(b)
            if b < REM:
                issue_idx(b, NROUNDS * NBUF + b)
        for b in range(REM):
            wait_idx(b)
            issue_gather(b)
        for b in range(REM):
            wait_gather(b)
            issue_scatter(b)
        for b in range(REM):
            wait_scatter(b)
        plsc.subcore_barrier()
        pltpu.sync_copy(acc.at[pl.ds(s * ZROWS, ZROWS)],
                        out.at[pl.ds(s * ZROWS, ZROWS)])

    @pl.when(c == 0)
    def _():
        run(tlo, lo_out)

    @pl.when(c == 1)
    def _():
        run(thi, hi_out)


@functools.cache
def _edge_agg_kernel():
    out = (
        jax.ShapeDtypeStruct((NAGG, HALF), jnp.float32),
        jax.ShapeDtypeStruct((NAGG, HALF), jnp.float32),
    )
    scratch = [
        pltpu.VMEM((NBUF, 2, K), jnp.int32),
        pltpu.VMEM((NBUF, K, HALF), jnp.float32),
        pltpu.VMEM_SHARED((NAGG, HALF), jnp.float32),
    ] + [pltpu.SemaphoreType.DMA] * (3 * NBUF)
    return pl.kernel(
        _edge_agg_body,
        out_type=out,
        mesh=_sc_mesh(),
        scratch_types=scratch,
    )


def _sc_edge_agg(tlo, thi, e4, zeros):
    return _edge_agg_kernel()(tlo, thi, e4, zeros)


RB = 400


def _linear_relu_body_split(lo_ref, hi_ref, w_ref, b_ref, olo_ref, ohi_ref):
    x = jnp.concatenate([lo_ref[...], hi_ref[...]], axis=1)
    y = jnp.dot(x, w_ref[...], preferred_element_type=jnp.float32) + b_ref[...]
    y = jnp.maximum(y, 0.0)
    olo_ref[...] = y[:, :HALF]
    ohi_ref[...] = y[:, HALF:]


def _tc_linear_relu_split(lo, hi, W, b2d):
    grid = (N_NODES // RB,)
    return pl.pallas_call(
        _linear_relu_body_split,
        grid=grid,
        in_specs=[
            pl.BlockSpec((RB, HALF), lambda i: (i, 0)),
            pl.BlockSpec((RB, HALF), lambda i: (i, 0)),
            pl.BlockSpec((D_IN, D_IN), lambda i: (0, 0)),
            pl.BlockSpec((1, D_IN), lambda i: (0, 0)),
        ],
        out_specs=(
            pl.BlockSpec((RB, HALF), lambda i: (i, 0)),
            pl.BlockSpec((RB, HALF), lambda i: (i, 0)),
        ),
        out_shape=(
            jax.ShapeDtypeStruct((NPAD, HALF), jnp.float32),
            jax.ShapeDtypeStruct((NPAD, HALF), jnp.float32),
        ),
    )(lo, hi, W, b2d)


def _linear_relu_body_full(lo_ref, hi_ref, w_ref, b_ref, o_ref):
    x = jnp.concatenate([lo_ref[...], hi_ref[...]], axis=1)
    y = jnp.dot(x, w_ref[...], preferred_element_type=jnp.float32) + b_ref[...]
    o_ref[...] = jnp.maximum(y, 0.0)


def _tc_linear_relu_full(lo, hi, W, b2d):
    grid = (N_NODES // RB,)
    return pl.pallas_call(
        _linear_relu_body_full,
        grid=grid,
        in_specs=[
            pl.BlockSpec((RB, HALF), lambda i: (i, 0)),
            pl.BlockSpec((RB, HALF), lambda i: (i, 0)),
            pl.BlockSpec((D_IN, D_IN), lambda i: (0, 0)),
            pl.BlockSpec((1, D_IN), lambda i: (0, 0)),
        ],
        out_specs=pl.BlockSpec((RB, D_IN), lambda i: (i, 0)),
        out_shape=jax.ShapeDtypeStruct((N_NODES, D_IN), jnp.float32),
    )(lo, hi, W, b2d)


def kernel(cncpt_ids, edge_index, emb_table, W1, b1, W2, b2):
    ids = cncpt_ids.astype(jnp.int32)
    ids_pad = jnp.concatenate(
        [ids, jnp.zeros((NPAD - N_NODES,), jnp.int32)])
    ei = edge_index.astype(jnp.int32)
    e4 = jnp.stack([ei[0].reshape(NS, CPT, K),
                    ei[1].reshape(NS, CPT, K)], axis=2)
    zeros = jnp.zeros((ZROWS, HALF), jnp.float32)

    flo, fhi = _sc_emb_gather(ids_pad, emb_table)

    Ws = jnp.stack([W1, W2])
    bs = jnp.stack([b1.reshape(1, D_IN), b2.reshape(1, D_IN)])

    def step(carry, wb):
        tlo, thi = carry
        W, b2d = wb
        alo, ahi = _sc_edge_agg(tlo, thi, e4, zeros)
        nlo, nhi = _tc_linear_relu_split(alo, ahi, W, b2d)
        return (nlo, nhi), None

    (olo, ohi), _ = lax.scan(step, (flo, fhi), (Ws, bs))
    return jnp.concatenate([olo[:N_NODES], ohi[:N_NODES]], axis=1)


# speedup vs baseline: 6.5602x; 1.0080x over previous
import functools

import jax
import jax.numpy as jnp
from jax import lax
from jax.experimental import pallas as pl
from jax.experimental.pallas import tpu as pltpu
from jax.experimental.pallas import tpu_sc as plsc

N_NODES = 10000
N_EDGES = 160000
D_IN = 256
HALF = 128

NC = 2
NS = 16
NW = NC * NS

NPAD = 10240
ROWS_W = NPAD // NW
GCHUNK = 80
GCHUNKS = ROWS_W // GCHUNK

EPT = N_EDGES // NS
K = 40
CPT = EPT // K
NAGG = 10240
ZROWS = NAGG // NS


def _sc_mesh():
    return plsc.VectorSubcoreMesh(core_axis_name="c", subcore_axis_name="s")


def _emb_gather_body(ids_hbm, emb_hbm, lo_hbm, hi_hbm, idx_v, stag, sem):
    c = lax.axis_index("c")
    s = lax.axis_index("s")
    wid = s * NC + c

    def chunk(j, carry):
        base = wid * ROWS_W + j * GCHUNK
        pltpu.sync_copy(ids_hbm.at[pl.ds(base, GCHUNK)], idx_v)
        pltpu.async_copy(emb_hbm.at[idx_v], stag, sem).wait()
        pltpu.sync_copy(stag.at[:, pl.ds(0, HALF)], lo_hbm.at[pl.ds(base, GCHUNK)])
        pltpu.sync_copy(stag.at[:, pl.ds(HALF, HALF)], hi_hbm.at[pl.ds(base, GCHUNK)])
        return carry

    lax.fori_loop(0, GCHUNKS, chunk, 0)


def _sc_emb_gather(ids_pad, emb_table):
    out = (
        jax.ShapeDtypeStruct((NPAD, HALF), jnp.float32),
        jax.ShapeDtypeStruct((NPAD, HALF), jnp.float32),
    )
    scratch = [
        pltpu.VMEM((GCHUNK,), jnp.int32),
        pltpu.VMEM((GCHUNK, D_IN), jnp.float32),
        pltpu.SemaphoreType.DMA,
    ]
    f = pl.kernel(
        _emb_gather_body,
        out_type=out,
        mesh=_sc_mesh(),
        scratch_types=scratch,
    )
    return f(ids_pad, emb_table)


NBUF = 8
NROUNDS = CPT // NBUF
REM = CPT - NROUNDS * NBUF


def _edge_agg_body(tlo, thi, e4, zeros_hbm, lo_out, hi_out,
                   ering, stags, acc, *sems):
    isems = [sems[b] for b in range(NBUF)]
    gsems = [sems[NBUF + b] for b in range(NBUF)]
    ssems = [sems[2 * NBUF + b] for b in range(NBUF)]
    c = lax.axis_index("c")
    s = lax.axis_index("s")

    pltpu.sync_copy(zeros_hbm, acc.at[pl.ds(s * ZROWS, ZROWS)])
    plsc.subcore_barrier()

    def run(table, out):
        def issue_idx(b, j):
            pltpu.async_copy(e4.at[s, j], ering.at[b], isems[b])

        def wait_idx(b):
            pltpu.make_async_copy(e4.at[s, 0], ering.at[b], isems[b]).wait()

        def issue_gather(b):
            pltpu.async_copy(table.at[ering.at[b, 0]], stags.at[b], gsems[b])

        def wait_gather(b):
            pltpu.make_async_copy(
                table.at[ering.at[b, 0]], stags.at[b], gsems[b]).wait()

        def issue_scatter(b):
            pltpu.async_copy(stags.at[b], acc.at[ering.at[b, 1]],
                             ssems[b], add=True)

        def wait_scatter(b):
            pltpu.make_async_copy(
                stags.at[b], acc.at[ering.at[b, 1]], ssems[b]).wait()

        for b in range(NBUF):
            issue_idx(b, b)
        for b in range(NBUF):
            wait_idx(b)
            issue_gather(b)

        def outer(t, carry):
            for b in range(NBUF):
                wait_gather(b)
                issue_scatter(b)
            for b in range(NBUF):
                j2 = jnp.minimum((t + 1) * NBUF + b, CPT - 1)
                wait_scatter(b)
                issue_idx(b, j2)
            for b in range(NBUF):
                wait_idx(b)
                issue_gather(b)
            return carry

        lax.fori_loop(0, NROUNDS - 1, outer, 0)
        for b in range(NBUF):
            wait_gather(b)
            issue_scatter(b)
        for b in range(NBUF):
            wait_scatter(b)
            if b < REM:
                issue_idx(b, NROUNDS * NBUF + b)
        for b in range(REM):
            wait_idx(b)
            issue_gather(b)
        for b in range(REM):
            wait_gather(b)
            issue_scatter(b)
        for b in range(REM):
            wait_scatter(b)
        plsc.subcore_barrier()
        pltpu.sync_copy(acc.at[pl.ds(s * ZROWS, ZROWS)],
                        out.at[pl.ds(s * ZROWS, ZROWS)])

    @pl.when(c == 0)
    def _():
        run(tlo, lo_out)

    @pl.when(c == 1)
    def _():
        run(thi, hi_out)


@functools.cache
def _edge_agg_kernel():
    out = (
        jax.ShapeDtypeStruct((NAGG, HALF), jnp.float32),
        jax.ShapeDtypeStruct((NAGG, HALF), jnp.float32),
    )
    scratch = [
        pltpu.VMEM((NBUF, 2, K), jnp.int32),
        pltpu.VMEM((NBUF, K, HALF), jnp.float32),
        pltpu.VMEM_SHARED((NAGG, HALF), jnp.float32),
    ] + [pltpu.SemaphoreType.DMA] * (3 * NBUF)
    return pl.kernel(
        _edge_agg_body,
        out_type=out,
        mesh=_sc_mesh(),
        scratch_types=scratch,
    )


def _sc_edge_agg(tlo, thi, e4, zeros):
    return _edge_agg_kernel()(tlo, thi, e4, zeros)


RB = 400


def _linear_relu_body_split(lo_ref, hi_ref, w_ref, b_ref, olo_ref, ohi_ref):
    x = jnp.concatenate([lo_ref[...], hi_ref[...]], axis=1)
    y = jnp.dot(x, w_ref[...], preferred_element_type=jnp.float32) + b_ref[...]
    y = jnp.maximum(y, 0.0)
    olo_ref[...] = y[:, :HALF]
    ohi_ref[...] = y[:, HALF:]


def _tc_linear_relu_split(lo, hi, W, b2d):
    grid = (N_NODES // RB,)
    return pl.pallas_call(
        _linear_relu_body_split,
        grid=grid,
        in_specs=[
            pl.BlockSpec((RB, HALF), lambda i: (i, 0)),
            pl.BlockSpec((RB, HALF), lambda i: (i, 0)),
            pl.BlockSpec((D_IN, D_IN), lambda i: (0, 0)),
            pl.BlockSpec((1, D_IN), lambda i: (0, 0)),
        ],
        out_specs=(
            pl.BlockSpec((RB, HALF), lambda i: (i, 0)),
            pl.BlockSpec((RB, HALF), lambda i: (i, 0)),
        ),
        out_shape=(
            jax.ShapeDtypeStruct((NPAD, HALF), jnp.float32),
            jax.ShapeDtypeStruct((NPAD, HALF), jnp.float32),
        ),
    )(lo, hi, W, b2d)


def _linear_relu_body_full(lo_ref, hi_ref, w_ref, b_ref, o_ref):
    x = jnp.concatenate([lo_ref[...], hi_ref[...]], axis=1)
    y = jnp.dot(x, w_ref[...], preferred_element_type=jnp.float32) + b_ref[...]
    o_ref[...] = jnp.maximum(y, 0.0)


def _tc_linear_relu_full(lo, hi, W, b2d):
    grid = (N_NODES // RB,)
    return pl.pallas_call(
        _linear_relu_body_full,
        grid=grid,
        in_specs=[
            pl.BlockSpec((RB, HALF), lambda i: (i, 0)),
            pl.BlockSpec((RB, HALF), lambda i: (i, 0)),
            pl.BlockSpec((D_IN, D_IN), lambda i: (0, 0)),
            pl.BlockSpec((1, D_IN), lambda i: (0, 0)),
        ],
        out_specs=pl.BlockSpec((RB, D_IN), lambda i: (i, 0)),
        out_shape=jax.ShapeDtypeStruct((N_NODES, D_IN), jnp.float32),
    )(lo, hi, W, b2d)


def kernel(cncpt_ids, edge_index, emb_table, W1, b1, W2, b2):
    ids = cncpt_ids.astype(jnp.int32)
    ids_pad = jnp.concatenate(
        [ids, jnp.zeros((NPAD - N_NODES,), jnp.int32)])
    ei = edge_index.astype(jnp.int32)
    e4 = jnp.stack([ei[0].reshape(NS, CPT, K),
                    ei[1].reshape(NS, CPT, K)], axis=2)
    zeros = jnp.zeros((ZROWS, HALF), jnp.float32)

    flo, fhi = _sc_emb_gather(ids_pad, emb_table)

    Ws = jnp.stack([W1, W2])
    bs = jnp.stack([b1.reshape(1, D_IN), b2.reshape(1, D_IN)])

    def step(carry, wb):
        tlo, thi = carry
        W, b2d = wb
        alo, ahi = _sc_edge_agg(tlo, thi, e4, zeros)
        nlo, nhi = _tc_linear_relu_split(alo, ahi, W, b2d)
        return (nlo, nhi), None

    (olo, ohi), _ = lax.scan(step, (flo, fhi), (Ws, bs))
    return jnp.concatenate([olo[:N_NODES], ohi[:N_NODES]], axis=1)


# speedup vs baseline: 7.0364x; 1.0726x over previous
import functools

import jax
import jax.numpy as jnp
from jax import lax
from jax.experimental import pallas as pl
from jax.experimental.pallas import tpu as pltpu
from jax.experimental.pallas import tpu_sc as plsc

N_NODES = 10000
N_EDGES = 160000
D_IN = 256
HALF = 128

NC = 2
NS = 16
NW = NC * NS

NPAD = 10240
ROWS_W = NPAD // NW
GCHUNK = 80
GCHUNKS = ROWS_W // GCHUNK

EPT = N_EDGES // NS
K = 40
CPT = EPT // K
NAGG = 10240
ZROWS = NAGG // NS


def _sc_mesh():
    return plsc.VectorSubcoreMesh(core_axis_name="c", subcore_axis_name="s")


def _emb_gather_body(ids_hbm, emb_hbm, lo_hbm, hi_hbm, idxs, stags, *sems):
    isems = [sems[0], sems[1]]
    gsems = [sems[2], sems[3]]
    wsems = [sems[4], sems[5]]
    c = lax.axis_index("c")
    s = lax.axis_index("s")
    wid = s * NC + c

    def base(j):
        return wid * ROWS_W + j * GCHUNK

    def issue_idx(b, j):
        pltpu.async_copy(ids_hbm.at[pl.ds(base(j), GCHUNK)], idxs.at[b],
                         isems[b])

    def wait_idx(b):
        pltpu.make_async_copy(ids_hbm.at[pl.ds(base(0), GCHUNK)], idxs.at[b],
                              isems[b]).wait()

    def issue_gather(b):
        pltpu.async_copy(emb_hbm.at[idxs.at[b]], stags.at[b], gsems[b])

    def wait_gather(b):
        pltpu.make_async_copy(emb_hbm.at[idxs.at[b]], stags.at[b],
                              gsems[b]).wait()

    def issue_write(b, j):
        pltpu.async_copy(stags.at[b, :, pl.ds(0, HALF)],
                         lo_hbm.at[pl.ds(base(j), GCHUNK)], wsems[b])
        pltpu.async_copy(stags.at[b, :, pl.ds(HALF, HALF)],
                         hi_hbm.at[pl.ds(base(j), GCHUNK)], wsems[b])

    def wait_write(b):
        pltpu.make_async_copy(stags.at[b, :, pl.ds(0, HALF)],
                              lo_hbm.at[pl.ds(base(0), GCHUNK)],
                              wsems[b]).wait()
        pltpu.make_async_copy(stags.at[b, :, pl.ds(HALF, HALF)],
                              hi_hbm.at[pl.ds(base(0), GCHUNK)],
                              wsems[b]).wait()

    issue_idx(0, 0)
    issue_idx(1, 1)
    wait_idx(0); issue_gather(0)
    wait_idx(1); issue_gather(1)
    wait_gather(0); issue_write(0, 0)
    wait_gather(1); issue_write(1, 1)
    wait_write(0); issue_idx(0, 2)
    wait_write(1); issue_idx(1, 3)
    wait_idx(0); issue_gather(0)
    wait_idx(1); issue_gather(1)
    wait_gather(0); issue_write(0, 2)
    wait_gather(1); issue_write(1, 3)
    wait_write(0)
    wait_write(1)


def _sc_emb_gather(ids_pad, emb_table):
    out = (
        jax.ShapeDtypeStruct((NPAD, HALF), jnp.float32),
        jax.ShapeDtypeStruct((NPAD, HALF), jnp.float32),
    )
    scratch = [
        pltpu.VMEM((2, GCHUNK), jnp.int32),
        pltpu.VMEM((2, GCHUNK, D_IN), jnp.float32),
    ] + [pltpu.SemaphoreType.DMA] * 6
    f = pl.kernel(
        _emb_gather_body,
        out_type=out,
        mesh=_sc_mesh(),
        scratch_types=scratch,
    )
    return f(ids_pad, emb_table)


NBUF = 8
NROUNDS = CPT // NBUF
REM = CPT - NROUNDS * NBUF


def _edge_agg_body(tlo, thi, e4, zeros_hbm, lo_out, hi_out,
                   ering, stags, acc, *sems):
    isems = [sems[b] for b in range(NBUF)]
    gsems = [sems[NBUF + b] for b in range(NBUF)]
    ssems = [sems[2 * NBUF + b] for b in range(NBUF)]
    c = lax.axis_index("c")
    s = lax.axis_index("s")

    pltpu.sync_copy(zeros_hbm, acc.at[pl.ds(s * ZROWS, ZROWS)])
    plsc.subcore_barrier()

    def run(table, out):
        def issue_idx(b, j):
            pltpu.async_copy(e4.at[s, j], ering.at[b], isems[b])

        def wait_idx(b):
            pltpu.make_async_copy(e4.at[s, 0], ering.at[b], isems[b]).wait()

        def issue_gather(b):
            pltpu.async_copy(table.at[ering.at[b, 0]], stags.at[b], gsems[b])

        def wait_gather(b):
            pltpu.make_async_copy(
                table.at[ering.at[b, 0]], stags.at[b], gsems[b]).wait()

        def issue_scatter(b):
            pltpu.async_copy(stags.at[b], acc.at[ering.at[b, 1]],
                             ssems[b], add=True)

        def wait_scatter(b):
            pltpu.make_async_copy(
                stags.at[b], acc.at[ering.at[b, 1]], ssems[b]).wait()

        for b in range(NBUF):
            issue_idx(b, b)
        for b in range(NBUF):
            wait_idx(b)
            issue_gather(b)

        def outer(t, carry):
            for b in range(NBUF):
                wait_gather(b)
                issue_scatter(b)
            for b in range(NBUF):
                j2 = jnp.minimum((t + 1) * NBUF + b, CPT - 1)
                wait_scatter(b)
                issue_idx(b, j2)
            for b in range(NBUF):
                wait_idx(b)
                issue_gather(b)
            return carry

        lax.fori_loop(0, NROUNDS - 1, outer, 0)
        for b in range(NBUF):
            wait_gather(b)
            issue_scatter(b)
        for b in range(NBUF):
            wait_scatter(b)
            if b < REM:
                issue_idx(b, NROUNDS * NBUF + b)
        for b in range(REM):
            wait_idx(b)
            issue_gather(b)
        for b in range(REM):
            wait_gather(b)
            issue_scatter(b)
        for b in range(REM):
            wait_scatter(b)
        plsc.subcore_barrier()
        pltpu.sync_copy(acc.at[pl.ds(s * ZROWS, ZROWS)],
                        out.at[pl.ds(s * ZROWS, ZROWS)])

    @pl.when(c == 0)
    def _():
        run(tlo, lo_out)

    @pl.when(c == 1)
    def _():
        run(thi, hi_out)


@functools.cache
def _edge_agg_kernel():
    out = (
        jax.ShapeDtypeStruct((NAGG, HALF), jnp.float32),
        jax.ShapeDtypeStruct((NAGG, HALF), jnp.float32),
    )
    scratch = [
        pltpu.VMEM((NBUF, 2, K), jnp.int32),
        pltpu.VMEM((NBUF, K, HALF), jnp.float32),
        pltpu.VMEM_SHARED((NAGG, HALF), jnp.float32),
    ] + [pltpu.SemaphoreType.DMA] * (3 * NBUF)
    return pl.kernel(
        _edge_agg_body,
        out_type=out,
        mesh=_sc_mesh(),
        scratch_types=scratch,
    )


def _sc_edge_agg(tlo, thi, e4, zeros):
    return _edge_agg_kernel()(tlo, thi, e4, zeros)


RB = 2000


def _linear_relu_body_split(lo_ref, hi_ref, w_ref, b_ref, olo_ref, ohi_ref):
    x = jnp.concatenate([lo_ref[...], hi_ref[...]], axis=1)
    y = jnp.dot(x, w_ref[...], preferred_element_type=jnp.float32) + b_ref[...]
    y = jnp.maximum(y, 0.0)
    olo_ref[...] = y[:, :HALF]
    ohi_ref[...] = y[:, HALF:]


def _tc_linear_relu_split(lo, hi, W, b2d):
    grid = (N_NODES // RB,)
    return pl.pallas_call(
        _linear_relu_body_split,
        grid=grid,
        in_specs=[
            pl.BlockSpec((RB, HALF), lambda i: (i, 0)),
            pl.BlockSpec((RB, HALF), lambda i: (i, 0)),
            pl.BlockSpec((D_IN, D_IN), lambda i: (0, 0)),
            pl.BlockSpec((1, D_IN), lambda i: (0, 0)),
        ],
        out_specs=(
            pl.BlockSpec((RB, HALF), lambda i: (i, 0)),
            pl.BlockSpec((RB, HALF), lambda i: (i, 0)),
        ),
        out_shape=(
            jax.ShapeDtypeStruct((NPAD, HALF), jnp.float32),
            jax.ShapeDtypeStruct((NPAD, HALF), jnp.float32),
        ),
    )(lo, hi, W, b2d)


def _linear_relu_body_full(lo_ref, hi_ref, w_ref, b_ref, o_ref):
    x = jnp.concatenate([lo_ref[...], hi_ref[...]], axis=1)
    y = jnp.dot(x, w_ref[...], preferred_element_type=jnp.float32) + b_ref[...]
    o_ref[...] = jnp.maximum(y, 0.0)


def _tc_linear_relu_full(lo, hi, W, b2d):
    grid = (N_NODES // RB,)
    return pl.pallas_call(
        _linear_relu_body_full,
        grid=grid,
        in_specs=[
            pl.BlockSpec((RB, HALF), lambda i: (i, 0)),
            pl.BlockSpec((RB, HALF), lambda i: (i, 0)),
            pl.BlockSpec((D_IN, D_IN), lambda i: (0, 0)),
            pl.BlockSpec((1, D_IN), lambda i: (0, 0)),
        ],
        out_specs=pl.BlockSpec((RB, D_IN), lambda i: (i, 0)),
        out_shape=jax.ShapeDtypeStruct((N_NODES, D_IN), jnp.float32),
    )(lo, hi, W, b2d)


def kernel(cncpt_ids, edge_index, emb_table, W1, b1, W2, b2):
    ids = cncpt_ids.astype(jnp.int32)
    ids_pad = jnp.concatenate(
        [ids, jnp.zeros((NPAD - N_NODES,), jnp.int32)])
    ei = edge_index.astype(jnp.int32)
    e4 = jnp.stack([ei[0].reshape(NS, CPT, K),
                    ei[1].reshape(NS, CPT, K)], axis=2)
    zeros = jnp.zeros((ZROWS, HALF), jnp.float32)

    flo, fhi = _sc_emb_gather(ids_pad, emb_table)

    Ws = jnp.stack([W1, W2])
    bs = jnp.stack([b1.reshape(1, D_IN), b2.reshape(1, D_IN)])

    def step(carry, wb):
        tlo, thi = carry
        W, b2d = wb
        alo, ahi = _sc_edge_agg(tlo, thi, e4, zeros)
        nlo, nhi = _tc_linear_relu_split(alo, ahi, W, b2d)
        return (nlo, nhi), None

    (olo, ohi), _ = lax.scan(step, (flo, fhi), (Ws, bs))
    return jnp.concatenate([olo[:N_NODES], ohi[:N_NODES]], axis=1)


# speedup vs baseline: 7.0406x; 1.0006x over previous
import functools

import jax
import jax.numpy as jnp
from jax import lax
from jax.experimental import pallas as pl
from jax.experimental.pallas import tpu as pltpu
from jax.experimental.pallas import tpu_sc as plsc

N_NODES = 10000
N_EDGES = 160000
D_IN = 256
HALF = 128

NC = 2
NS = 16
NW = NC * NS

NPAD = 10240
ROWS_W = NPAD // NW
GCHUNK = 80
GCHUNKS = ROWS_W // GCHUNK

EPT = N_EDGES // NS
K = 40
CPT = EPT // K
NAGG = 10240
ZROWS = NAGG // NS


def _sc_mesh():
    return plsc.VectorSubcoreMesh(core_axis_name="c", subcore_axis_name="s")


def _emb_gather_body(ids_hbm, emb_hbm, lo_hbm, hi_hbm, idxs, stags, *sems):
    isems = [sems[0], sems[1]]
    gsems = [sems[2], sems[3]]
    wsems = [sems[4], sems[5]]
    c = lax.axis_index("c")
    s = lax.axis_index("s")
    wid = s * NC + c

    def base(j):
        return wid * ROWS_W + j * GCHUNK

    def issue_idx(b, j):
        pltpu.async_copy(ids_hbm.at[pl.ds(base(j), GCHUNK)], idxs.at[b],
                         isems[b])

    def wait_idx(b):
        pltpu.make_async_copy(ids_hbm.at[pl.ds(base(0), GCHUNK)], idxs.at[b],
                              isems[b]).wait()

    def issue_gather(b):
        pltpu.async_copy(emb_hbm.at[idxs.at[b]], stags.at[b], gsems[b])

    def wait_gather(b):
        pltpu.make_async_copy(emb_hbm.at[idxs.at[b]], stags.at[b],
                              gsems[b]).wait()

    def issue_write(b, j):
        pltpu.async_copy(stags.at[b, :, pl.ds(0, HALF)],
                         lo_hbm.at[pl.ds(base(j), GCHUNK)], wsems[b])
        pltpu.async_copy(stags.at[b, :, pl.ds(HALF, HALF)],
                         hi_hbm.at[pl.ds(base(j), GCHUNK)], wsems[b])

    def wait_write(b):
        pltpu.make_async_copy(stags.at[b, :, pl.ds(0, HALF)],
                              lo_hbm.at[pl.ds(base(0), GCHUNK)],
                              wsems[b]).wait()
        pltpu.make_async_copy(stags.at[b, :, pl.ds(HALF, HALF)],
                              hi_hbm.at[pl.ds(base(0), GCHUNK)],
                              wsems[b]).wait()

    issue_idx(0, 0)
    issue_idx(1, 1)
    wait_idx(0); issue_gather(0)
    wait_idx(1); issue_gather(1)
    wait_gather(0); issue_write(0, 0)
    wait_gather(1); issue_write(1, 1)
    wait_write(0); issue_idx(0, 2)
    wait_write(1); issue_idx(1, 3)
    wait_idx(0); issue_gather(0)
    wait_idx(1); issue_gather(1)
    wait_gather(0); issue_write(0, 2)
    wait_gather(1); issue_write(1, 3)
    wait_write(0)
    wait_write(1)


def _sc_emb_gather(ids_pad, emb_table):
    out = (
        jax.ShapeDtypeStruct((NPAD, HALF), jnp.float32),
        jax.ShapeDtypeStruct((NPAD, HALF), jnp.float32),
    )
    scratch = [
        pltpu.VMEM((2, GCHUNK), jnp.int32),
        pltpu.VMEM((2, GCHUNK, D_IN), jnp.float32),
    ] + [pltpu.SemaphoreType.DMA] * 6
    f = pl.kernel(
        _emb_gather_body,
        out_type=out,
        mesh=_sc_mesh(),
        scratch_types=scratch,
    )
    return f(ids_pad, emb_table)


NBUF = 8
NROUNDS = CPT // NBUF
REM = CPT - NROUNDS * NBUF


def _edge_agg_body(tlo, thi, e4, zeros_hbm, lo_out, hi_out,
                   ering, stags, acc, *sems):
    isems = [sems[b] for b in range(NBUF)]
    gsems = [sems[NBUF + b] for b in range(NBUF)]
    ssems = [sems[2 * NBUF + b] for b in range(NBUF)]
    c = lax.axis_index("c")
    s = lax.axis_index("s")

    pltpu.sync_copy(zeros_hbm, acc.at[pl.ds(s * ZROWS, ZROWS)])
    plsc.subcore_barrier()

    def run(table, out):
        def issue_idx(b, j):
            pltpu.async_copy(e4.at[s, j], ering.at[b], isems[b])

        def wait_idx(b):
            pltpu.make_async_copy(e4.at[s, 0], ering.at[b], isems[b]).wait()

        def issue_gather(b):
            pltpu.async_copy(table.at[ering.at[b, 0]], stags.at[b], gsems[b])

        def wait_gather(b):
            pltpu.make_async_copy(
                table.at[ering.at[b, 0]], stags.at[b], gsems[b]).wait()

        def issue_scatter(b):
            pltpu.async_copy(stags.at[b], acc.at[ering.at[b, 1]],
                             ssems[b], add=True)

        def wait_scatter(b):
            pltpu.make_async_copy(
                stags.at[b], acc.at[ering.at[b, 1]], ssems[b]).wait()

        for b in range(NBUF):
            issue_idx(b, b)
        for b in range(NBUF):
            wait_idx(b)
            issue_gather(b)

        def outer(t, carry):
            for b in range(NBUF):
                wait_gather(b)
                issue_scatter(b)
            for b in range(NBUF):
                j2 = jnp.minimum((t + 1) * NBUF + b, CPT - 1)
                wait_scatter(b)
                issue_idx(b, j2)
            for b in range(NBUF):
                wait_idx(b)
                issue_gather(b)
            return carry

        lax.fori_loop(0, NROUNDS - 1, outer, 0)
        for b in range(NBUF):
            wait_gather(b)
            issue_scatter(b)
        for b in range(NBUF):
            wait_scatter(b)
            if b < REM:
                issue_idx(b, NROUNDS * NBUF + b)
        for b in range(REM):
            wait_idx(b)
            issue_gather(b)
        for b in range(REM):
            wait_gather(b)
            issue_scatter(b)
        for b in range(REM):
            wait_scatter(b)
        plsc.subcore_barrier()
        pltpu.sync_copy(acc.at[pl.ds(s * ZROWS, ZROWS)],
                        out.at[pl.ds(s * ZROWS, ZROWS)])

    @pl.when(c == 0)
    def _():
        run(tlo, lo_out)

    @pl.when(c == 1)
    def _():
        run(thi, hi_out)


@functools.cache
def _edge_agg_kernel():
    out = (
        jax.ShapeDtypeStruct((NAGG, HALF), jnp.float32),
        jax.ShapeDtypeStruct((NAGG, HALF), jnp.float32),
    )
    scratch = [
        pltpu.VMEM((NBUF, 2, K), jnp.int32),
        pltpu.VMEM((NBUF, K, HALF), jnp.float32),
        pltpu.VMEM_SHARED((NAGG, HALF), jnp.float32),
    ] + [pltpu.SemaphoreType.DMA] * (3 * NBUF)
    return pl.kernel(
        _edge_agg_body,
        out_type=out,
        mesh=_sc_mesh(),
        scratch_types=scratch,
    )


def _sc_edge_agg(tlo, thi, e4, zeros):
    return _edge_agg_kernel()(tlo, thi, e4, zeros)


RB = 2000


def _linear_relu_body_split(lo_ref, hi_ref, w_ref, b_ref, olo_ref, ohi_ref):
    x = jnp.concatenate([lo_ref[...], hi_ref[...]], axis=1)
    y = jnp.dot(x, w_ref[...], preferred_element_type=jnp.float32) + b_ref[...]
    y = jnp.maximum(y, 0.0)
    olo_ref[...] = y[:, :HALF]
    ohi_ref[...] = y[:, HALF:]


def _tc_linear_relu_split(lo, hi, W, b2d):
    grid = (N_NODES // RB,)
    return pl.pallas_call(
        _linear_relu_body_split,
        grid=grid,
        in_specs=[
            pl.BlockSpec((RB, HALF), lambda i: (i, 0)),
            pl.BlockSpec((RB, HALF), lambda i: (i, 0)),
            pl.BlockSpec((D_IN, D_IN), lambda i: (0, 0)),
            pl.BlockSpec((1, D_IN), lambda i: (0, 0)),
        ],
        out_specs=(
            pl.BlockSpec((RB, HALF), lambda i: (i, 0)),
            pl.BlockSpec((RB, HALF), lambda i: (i, 0)),
        ),
        out_shape=(
            jax.ShapeDtypeStruct((NPAD, HALF), jnp.float32),
            jax.ShapeDtypeStruct((NPAD, HALF), jnp.float32),
        ),
    )(lo, hi, W, b2d)


def kernel(cncpt_ids, edge_index, emb_table, W1, b1, W2, b2):
    ids = cncpt_ids.astype(jnp.int32)
    ids_pad = jnp.concatenate(
        [ids, jnp.zeros((NPAD - N_NODES,), jnp.int32)])
    ei = edge_index.astype(jnp.int32)
    e4 = jnp.stack([ei[0].reshape(NS, CPT, K),
                    ei[1].reshape(NS, CPT, K)], axis=2)
    zeros = jnp.zeros((ZROWS, HALF), jnp.float32)

    flo, fhi = _sc_emb_gather(ids_pad, emb_table)

    Ws = jnp.stack([W1, W2])
    bs = jnp.stack([b1.reshape(1, D_IN), b2.reshape(1, D_IN)])

    def step(carry, wb):
        tlo, thi = carry
        W, b2d = wb
        alo, ahi = _sc_edge_agg(tlo, thi, e4, zeros)
        nlo, nhi = _tc_linear_relu_split(alo, ahi, W, b2d)
        return (nlo, nhi), None

    (olo, ohi), _ = lax.scan(step, (flo, fhi), (Ws, bs))
    return jnp.concatenate([olo[:N_NODES], ohi[:N_NODES]], axis=1)
